# baseline (speedup 1.0000x reference)
import functools

import jax
import jax.numpy as jnp
from jax import lax
from jax.experimental import pallas as pl
from jax.experimental.pallas import tpu as pltpu
from jax.experimental.pallas import tpu_sc as plsc

_B, _T, _D = 4, 2048, 1024
_NC, _NS = 2, 16
_NW = _NC * _NS
_TPW = _T // _NW
_R = 16
_NSUB = _TPW // _R
_CHUNK = _R * _D
_NVEC = _CHUNK // 16
_UNROLL = 8


def _sc_body(feat_hbm, sin_hbm, out_hbm, sin_v, feat_v):
    wid = lax.axis_index("s") * _NC + lax.axis_index("c")
    t0 = wid * _TPW

    def add_block(kk, carry):
        for j in range(_UNROLL):
            sl = pl.ds((kk * _UNROLL + j) * 16, 16)
            feat_v[sl] = feat_v[sl] + sin_v[sl]
        return carry

    for s in range(_NSUB):
        sin_off = (t0 + s * _R) * _D
        pltpu.sync_copy(sin_hbm.at[pl.ds(sin_off, _CHUNK)], sin_v)
        for b in range(_B):
            off = b * _T * _D + sin_off
            pltpu.sync_copy(feat_hbm.at[pl.ds(off, _CHUNK)], feat_v)
            lax.fori_loop(0, _NVEC // _UNROLL, add_block, 0)
            pltpu.sync_copy(feat_v, out_hbm.at[pl.ds(off, _CHUNK)])


_sc_kernel = functools.partial(
    pl.kernel,
    out_type=jax.ShapeDtypeStruct((_B * _T * _D,), jnp.float32),
    mesh=plsc.VectorSubcoreMesh(core_axis_name="c", subcore_axis_name="s"),
    scratch_types=[
        pltpu.VMEM((_CHUNK,), jnp.float32),
        pltpu.VMEM((_CHUNK,), jnp.float32),
    ],
)(_sc_body)


def kernel(features, sinusoids):
    B, T, D = features.shape
    flat = _sc_kernel(features.reshape(-1), sinusoids.reshape(-1))
    return flat.reshape(B, T, D)


# speedup vs baseline: 1.2057x; 1.2057x over previous
import functools

import jax
import jax.numpy as jnp
from jax import lax
from jax.experimental import pallas as pl
from jax.experimental.pallas import tpu as pltpu
from jax.experimental.pallas import tpu_sc as plsc

_B, _T, _D = 4, 2048, 1024
_NC, _NS = 2, 16
_NW = _NC * _NS
_TPW = _T // _NW
_R = 16
_NSUB = _TPW // _R
_CHUNK = _R * _D
_NVEC = _CHUNK // 16
_UNROLL = 8


def _sc_body(feat_hbm, sin_hbm, out_hbm,
             fb0, fb1, sb0, sb1,
             si0, si1, so0, so1, ss0, ss1):
    wid = lax.axis_index("s") * _NC + lax.axis_index("c")
    t0 = wid * _TPW

    fbufs, sbufs = [fb0, fb1], [sb0, sb1]
    sems_in, sems_out, sems_sin = [si0, si1], [so0, so1], [ss0, ss1]

    def make_add(fb, sb):
        def add_block(kk, carry):
            for j in range(_UNROLL):
                sl = pl.ds((kk * _UNROLL + j) * 16, 16)
                plsc.addupdate(fb.at[sl], sb[sl])
            return carry
        return add_block

    adds = [[make_add(fb, sb) for sb in sbufs] for fb in fbufs]

    items = [(s, b) for s in range(_NSUB) for b in range(_B)]

    def sin_off(s):
        return (t0 + s * _R) * _D

    def feat_off(s, b):
        return b * _T * _D + sin_off(s)

    sin_h = [None] * _NSUB
    sin_h[0] = pltpu.async_copy(
        sin_hbm.at[pl.ds(sin_off(0), _CHUNK)], sbufs[0], sems_sin[0])
    in_h = [None, None]
    out_h = [None, None]
    in_h[0] = pltpu.async_copy(
        feat_hbm.at[pl.ds(feat_off(0, 0), _CHUNK)], fbufs[0], sems_in[0])

    for i, (s, b) in enumerate(items):
        p = i % 2
        q = 1 - p
        if i + 1 < len(items):
            s2, b2 = items[i + 1]
            if s2 != s:
                sin_h[s2] = pltpu.async_copy(
                    sin_hbm.at[pl.ds(sin_off(s2), _CHUNK)],
                    sbufs[s2 % 2], sems_sin[s2 % 2])
            if out_h[q] is not None:
                out_h[q].wait()
                out_h[q] = None
            in_h[q] = pltpu.async_copy(
                feat_hbm.at[pl.ds(feat_off(s2, b2), _CHUNK)],
                fbufs[q], sems_in[q])
        in_h[p].wait()
        if b == 0:
            sin_h[s].wait()
        lax.fori_loop(0, _NVEC // _UNROLL, adds[p][s % 2], 0)
        out_h[p] = pltpu.async_copy(
            fbufs[p], out_hbm.at[pl.ds(feat_off(s, b), _CHUNK)], sems_out[p])

    out_h[0].wait()
    out_h[1].wait()


_sc_kernel = functools.partial(
    pl.kernel,
    out_type=jax.ShapeDtypeStruct((_B * _T * _D,), jnp.float32),
    mesh=plsc.VectorSubcoreMesh(core_axis_name="c", subcore_axis_name="s"),
    scratch_types=[
        pltpu.VMEM((_CHUNK,), jnp.float32),
        pltpu.VMEM((_CHUNK,), jnp.float32),
        pltpu.VMEM((_CHUNK,), jnp.float32),
        pltpu.VMEM((_CHUNK,), jnp.float32),
        pltpu.SemaphoreType.DMA,
        pltpu.SemaphoreType.DMA,
        pltpu.SemaphoreType.DMA,
        pltpu.SemaphoreType.DMA,
        pltpu.SemaphoreType.DMA,
        pltpu.SemaphoreType.DMA,
    ],
)(_sc_body)


def kernel(features, sinusoids):
    B, T, D = features.shape
    flat = _sc_kernel(features.reshape(-1), sinusoids.reshape(-1))
    return flat.reshape(B, T, D)
